# baseline (speedup 1.0000x reference)
import jax
import jax.numpy as jnp
from jax.experimental import pallas as pl
from jax.experimental.pallas import tpu as pltpu

DIM = 1024
NS = 64
NB = 16
B = 4
CHUNK = 128


def _scan_kernel(x_ref, Wr_ref, Wkv_ref, Wb_ref, bb_ref, Wq_ref,
                 out_ref, sfin_ref,
                 S_ref, kn_ref, v_ref, be_ref, q_ref, smX_ref, smE_ref, smO_ref,
                 i1_ref, i2_ref, w1_ref, w2_ref):
    i = pl.program_id(0)

    @pl.when(i == 0)
    def _init():
        S_ref[...] = jnp.zeros_like(S_ref)

    cb = CHUNK * B
    xf = x_ref[...].reshape(cb, DIM)

    logits = jax.lax.dot_general(xf, Wr_ref[...], (((1,), (1,)), ((), ())),
                                 preferred_element_type=jnp.float32)
    sm = jax.nn.softmax(logits, axis=-1)
    iota = jax.lax.broadcasted_iota(jnp.int32, (cb, NB), 1)
    i1 = jnp.argmax(logits, axis=-1)
    sel1 = iota == i1[:, None]
    i2 = jnp.argmax(jnp.where(sel1, -jnp.inf, logits), axis=-1)
    sel2 = iota == i2[:, None]
    s1 = jnp.sum(jnp.where(sel1, sm, 0.0), axis=-1)
    s2 = jnp.sum(jnp.where(sel2, sm, 0.0), axis=-1)
    denom = s1 + s2 + 1e-8
    rowi = jax.lax.broadcasted_iota(jnp.int32, (NB, NB // 2), 0)
    colj = jax.lax.broadcasted_iota(jnp.int32, (NB, NB // 2), 1)
    selE = (rowi == 2 * colj).astype(jnp.float32)
    selO = (rowi == 2 * colj + 1).astype(jnp.float32)
    smE = jax.lax.dot_general(sm, selE, (((1,), (0,)), ((), ())),
                              preferred_element_type=jnp.float32)
    smO = jax.lax.dot_general(sm, selO, (((1,), (0,)), ((), ())),
                              preferred_element_type=jnp.float32)
    rowi2 = jax.lax.broadcasted_iota(jnp.int32, (NB, NB * NS), 0)
    colj2 = jax.lax.broadcasted_iota(jnp.int32, (NB, NB * NS), 1)
    selR = (colj2 // NS == rowi2).astype(jnp.float32)
    smrep = jax.lax.dot_general(sm, selR, (((1,), (0,)), ((), ())),
                                preferred_element_type=jnp.float32)
    smE_ref[...] = smE.reshape(CHUNK, B, NB // 2)
    smO_ref[...] = smO.reshape(CHUNK, B, NB // 2)
    i1_ref[...] = i1.astype(jnp.int32).reshape(CHUNK, B)
    i2_ref[...] = i2.astype(jnp.int32).reshape(CHUNK, B)
    w1_ref[...] = (s1 / denom).reshape(CHUNK, B)
    w2_ref[...] = (s2 / denom).reshape(CHUNK, B)

    kv = jax.lax.dot_general(xf, Wkv_ref[...], (((1,), (1,)), ((), ())),
                             preferred_element_type=jnp.float32)
    bm = jax.lax.dot_general(xf, Wb_ref[...], (((1,), (1,)), ((), ())),
                             preferred_element_type=jnp.float32)
    for n in range(NB):
        k_n = kv[:, 128 * n:128 * n + NS]
        nrm = jnp.sqrt(jnp.sum(k_n * k_n, axis=-1, keepdims=True)) + 1e-6
        kn_ref[:, :, n, :] = (k_n / nrm).reshape(CHUNK, B, NS)
        v_ref[:, :, n, :] = kv[:, 128 * n + NS:128 * n + 128].reshape(CHUNK, B, NS)
        b_n = bm[:, NS * n:NS * n + NS] + bb_ref[n][None, :]
        be_ref[:, :, n, :] = jax.nn.sigmoid(b_n).reshape(CHUNK, B, NS)
        smX_ref[:, :, n, :] = smrep[:, NS * n:NS * n + NS].reshape(CHUNK, B, NS)
    q = jax.lax.dot_general(xf, Wq_ref[...], (((1,), (1,)), ((), ())),
                            preferred_element_type=jnp.float32)
    q_ref[...] = q.reshape(CHUNK, B, NS)

    lane = jax.lax.broadcasted_iota(jnp.int32, (NS, 2 * NS), 1)

    def _f(z):
        return z * z * jax.nn.sigmoid(z)

    def body(t, carry):
        slots = []
        for b in range(B):
            slots.append((b, i1_ref[t, b], w1_ref[pl.ds(t, 1), pl.ds(b, 1)]))
            slots.append((b, i2_ref[t, b], w2_ref[pl.ds(t, 1), pl.ds(b, 1)]))
        S = S_ref[...]
        pair8 = [S_ref[b, n // 2] for b, n, _ in slots]
        S8 = jnp.stack([jnp.where((n % 2) == 0, p[:, :NS], p[:, NS:])
                        for p, (_, n, _) in zip(pair8, slots)])
        kn8 = jnp.stack([kn_ref[t, b, pl.ds(n, 1)] for b, n, _ in slots])
        v8 = jnp.stack([v_ref[t, b, pl.ds(n, 1)] for b, n, _ in slots])
        be8 = jnp.stack([be_ref[t, b, pl.ds(n, 1)] for b, n, _ in slots])
        w8 = jnp.stack([w[:, :, None] for _, _, w in slots])
        retr = jnp.sum(S8 * kn8, axis=-1, keepdims=True)
        delta = jnp.swapaxes(v8, -1, -2) - retr
        S_upd = jnp.tanh(jnp.swapaxes(be8, -1, -2) * S8 + delta * kn8)
        S_new8 = S8 + w8[:, 0] * (S_upd - S8)

        q_t = q_ref[t]
        qq = jnp.concatenate([q_t, q_t], -1)
        P = S * qq[:, None, None, :]
        z_lo = jnp.sum(P[..., :NS], axis=-1)
        z_hi = jnp.sum(P[..., NS:], axis=-1)
        zc = jnp.concatenate([z_lo, z_hi], -1)
        bo = _f(zc)
        smc = jnp.concatenate(
            [jnp.broadcast_to(smE_ref[t][..., None], (B, NB // 2, NS)),
             jnp.broadcast_to(smO_ref[t][..., None], (B, NB // 2, NS))], -1)
        ws = jnp.sum(bo * smc, axis=1)
        base_out = ws[:, :NS] + ws[:, NS:]

        q8 = jnp.stack([q_t[b:b + 1, :] for b, _, _ in slots])
        zo8 = jnp.sum(S8 * q8, axis=-1, keepdims=True)
        zn8 = jnp.sum(S_new8 * q8, axis=-1, keepdims=True)
        df8 = _f(zn8) - _f(zo8)
        sv8 = [smX_ref[t, b, pl.ds(n, 1)] for b, n, _ in slots]
        corr = [jnp.transpose(df8[2 * b]) * sv8[2 * b]
                + jnp.transpose(df8[2 * b + 1]) * sv8[2 * b + 1]
                for b in range(B)]
        out_ref[t] = base_out + jnp.concatenate(corr, axis=0)

        for b in range(B):
            k1, k2 = 2 * b, 2 * b + 1
            n1, n2 = slots[k1][1], slots[k2][1]
            m1, m2 = n1 // 2, n2 // 2
            msk1 = (lane < NS) == ((n1 % 2) == 0)
            msk2 = (lane < NS) == ((n2 % 2) == 0)
            wp1 = jnp.where(msk1,
                            jnp.concatenate([S_new8[k1], S_new8[k1]], -1),
                            pair8[k1])
            S_ref[b, m1] = wp1
            base2 = jnp.where(m1 == m2, wp1, pair8[k2])
            wp2 = jnp.where(msk2,
                            jnp.concatenate([S_new8[k2], S_new8[k2]], -1),
                            base2)
            S_ref[b, m2] = wp2
        return carry

    jax.lax.fori_loop(0, CHUNK, body, 0)

    @pl.when(i == pl.num_programs(0) - 1)
    def _fin():
        for n in range(NB):
            sfin_ref[:, n] = S_ref[:, n // 2, :, NS * (n % 2):NS * (n % 2) + NS]


def kernel(x, W_router, W_kv, W_beta, b_beta, W_q):
    Tlen, batch, dim = x.shape
    out, sfin = pl.pallas_call(
        _scan_kernel,
        grid=(Tlen // CHUNK,),
        in_specs=[
            pl.BlockSpec((CHUNK, B, DIM), lambda i: (i, 0, 0)),
            pl.BlockSpec((NB, DIM), lambda i: (0, 0)),
            pl.BlockSpec((NB * 2 * NS, DIM), lambda i: (0, 0)),
            pl.BlockSpec((NB * NS, DIM), lambda i: (0, 0)),
            pl.BlockSpec((NB, NS), lambda i: (0, 0)),
            pl.BlockSpec((NS, DIM), lambda i: (0, 0)),
        ],
        out_specs=[
            pl.BlockSpec((CHUNK, B, NS), lambda i: (i, 0, 0)),
            pl.BlockSpec((B, NB, NS, NS), lambda i: (0, 0, 0, 0)),
        ],
        out_shape=[
            jax.ShapeDtypeStruct((Tlen, B, NS), jnp.float32),
            jax.ShapeDtypeStruct((B, NB, NS, NS), jnp.float32),
        ],
        scratch_shapes=[
            pltpu.VMEM((B, NB // 2, NS, 2 * NS), jnp.float32),
            pltpu.VMEM((CHUNK, B, NB, NS), jnp.float32),
            pltpu.VMEM((CHUNK, B, NB, NS), jnp.float32),
            pltpu.VMEM((CHUNK, B, NB, NS), jnp.float32),
            pltpu.VMEM((CHUNK, B, NS), jnp.float32),
            pltpu.VMEM((CHUNK, B, NB, NS), jnp.float32),
            pltpu.VMEM((CHUNK, B, NB // 2), jnp.float32),
            pltpu.VMEM((CHUNK, B, NB // 2), jnp.float32),
            pltpu.VMEM((CHUNK, B), jnp.int32),
            pltpu.VMEM((CHUNK, B), jnp.int32),
            pltpu.VMEM((CHUNK, B), jnp.float32),
            pltpu.VMEM((CHUNK, B), jnp.float32),
        ],
    )(x, W_router, W_kv, W_beta, b_beta, W_q)
    return out, sfin


# speedup vs baseline: 1.2422x; 1.2422x over previous
import jax
import jax.numpy as jnp
from jax.experimental import pallas as pl
from jax.experimental.pallas import tpu as pltpu

DIM = 1024
NS = 64
NB = 16
B = 4
CHUNK = 256


def _scan_kernel(x_ref, Wr_ref, Wkv_ref, Wb_ref, bb_ref, Wq_ref,
                 out_ref, sfin_ref,
                 S_ref, kn_ref, be_ref, q_ref, smE_ref, smO_ref,
                 i1_ref, i2_ref, w1_ref, w2_ref):
    i = pl.program_id(0)

    @pl.when(i == 0)
    def _init():
        S_ref[...] = jnp.zeros_like(S_ref)

    cb = CHUNK * B
    xf = x_ref[...].reshape(cb, DIM)

    logits = jax.lax.dot_general(xf, Wr_ref[...], (((1,), (1,)), ((), ())),
                                 preferred_element_type=jnp.float32)
    sm = jax.nn.softmax(logits, axis=-1)
    iota = jax.lax.broadcasted_iota(jnp.int32, (cb, NB), 1)
    i1 = jnp.argmax(logits, axis=-1)
    sel1 = iota == i1[:, None]
    i2 = jnp.argmax(jnp.where(sel1, -jnp.inf, logits), axis=-1)
    sel2 = iota == i2[:, None]
    s1 = jnp.sum(jnp.where(sel1, sm, 0.0), axis=-1)
    s2 = jnp.sum(jnp.where(sel2, sm, 0.0), axis=-1)
    denom = s1 + s2 + 1e-8
    rowi = jax.lax.broadcasted_iota(jnp.int32, (NB, NB // 2), 0)
    colj = jax.lax.broadcasted_iota(jnp.int32, (NB, NB // 2), 1)
    selE = (rowi == 2 * colj).astype(jnp.float32)
    selO = (rowi == 2 * colj + 1).astype(jnp.float32)
    smE = jax.lax.dot_general(sm, selE, (((1,), (0,)), ((), ())),
                              preferred_element_type=jnp.float32)
    smO = jax.lax.dot_general(sm, selO, (((1,), (0,)), ((), ())),
                              preferred_element_type=jnp.float32)
    smE_ref[...] = smE.reshape(CHUNK, B, NB // 2)
    smO_ref[...] = smO.reshape(CHUNK, B, NB // 2)
    i1_ref[...] = i1.astype(jnp.int32).reshape(CHUNK, B)
    i2_ref[...] = i2.astype(jnp.int32).reshape(CHUNK, B)
    w1_ref[...] = (s1 / denom).reshape(CHUNK, B)
    w2_ref[...] = (s2 / denom).reshape(CHUNK, B)

    kv = jax.lax.dot_general(xf, Wkv_ref[...], (((1,), (1,)), ((), ())),
                             preferred_element_type=jnp.float32)
    bm = jax.lax.dot_general(xf, Wb_ref[...], (((1,), (1,)), ((), ())),
                             preferred_element_type=jnp.float32)
    for n in range(NB):
        k_n = kv[:, 128 * n:128 * n + NS]
        nrm = jnp.sqrt(jnp.sum(k_n * k_n, axis=-1, keepdims=True)) + 1e-6
        knv = jnp.concatenate([k_n / nrm, kv[:, 128 * n + NS:128 * n + 128]], -1)
        kn_ref[:, :, n, :] = knv.reshape(CHUNK, B, 2 * NS)
        b_n = bm[:, NS * n:NS * n + NS] + bb_ref[n][None, :]
        be_ref[:, :, n, :] = jax.nn.sigmoid(b_n).reshape(CHUNK, B, NS)
    q = jax.lax.dot_general(xf, Wq_ref[...], (((1,), (1,)), ((), ())),
                            preferred_element_type=jnp.float32)
    q_ref[...] = q.reshape(CHUNK, B, NS)

    lane = jax.lax.broadcasted_iota(jnp.int32, (NS, 2 * NS), 1)

    def body(t, carry):
        slots = []
        for b in range(B):
            slots.append((b, i1_ref[t, b], w1_ref[t, b]))
            slots.append((b, i2_ref[t, b], w2_ref[t, b]))
        pair8 = [S_ref[b, n // 2] for b, n, _ in slots]
        S8 = jnp.stack([jnp.where((n % 2) == 0, p[:, :NS], p[:, NS:])
                        for p, (_, n, _) in zip(pair8, slots)])
        knv8 = [kn_ref[t, b, pl.ds(n, 1)] for b, n, _ in slots]
        kn8 = jnp.stack([kv_[:, :NS] for kv_ in knv8])
        v8 = jnp.stack([kv_[:, NS:] for kv_ in knv8])
        be8 = jnp.stack([be_ref[t, b, pl.ds(n, 1)] for b, n, _ in slots])
        retr = jnp.sum(S8 * kn8, axis=-1, keepdims=True)
        delta = jnp.swapaxes(v8, -1, -2) - retr
        S_upd = jnp.tanh(jnp.swapaxes(be8, -1, -2) * S8 + delta * kn8)
        w8 = jnp.stack([jnp.full((1, 1), 1.0, jnp.float32) * w
                        for _, _, w in slots])
        S_new8 = S8 + w8 * (S_upd - S8)
        for b in range(B):
            k1, k2 = 2 * b, 2 * b + 1
            n1, n2 = slots[k1][1], slots[k2][1]
            m1, m2 = n1 // 2, n2 // 2
            msk1 = (lane < NS) == ((n1 % 2) == 0)
            msk2 = (lane < NS) == ((n2 % 2) == 0)
            wp1 = jnp.where(msk1,
                            jnp.concatenate([S_new8[k1], S_new8[k1]], -1),
                            pair8[k1])
            S_ref[b, m1] = wp1
            base2 = jnp.where(m1 == m2, wp1, pair8[k2])
            wp2 = jnp.where(msk2,
                            jnp.concatenate([S_new8[k2], S_new8[k2]], -1),
                            base2)
            S_ref[b, m2] = wp2
        S = S_ref[...]
        q_t = q_ref[t]
        qq = jnp.concatenate([q_t, q_t], -1)
        P = S * qq[:, None, None, :]
        z_lo = jnp.sum(P[..., :NS], axis=-1)
        z_hi = jnp.sum(P[..., NS:], axis=-1)
        zc = jnp.concatenate([z_lo, z_hi], -1)
        bo = zc * zc * jax.nn.sigmoid(zc)
        smc = jnp.concatenate(
            [jnp.broadcast_to(smE_ref[t][..., None], (B, NB // 2, NS)),
             jnp.broadcast_to(smO_ref[t][..., None], (B, NB // 2, NS))], -1)
        ws = jnp.sum(bo * smc, axis=1)
        out_ref[t] = ws[:, :NS] + ws[:, NS:]
        return carry

    jax.lax.fori_loop(0, CHUNK, body, 0)

    @pl.when(i == pl.num_programs(0) - 1)
    def _fin():
        for n in range(NB):
            sfin_ref[:, n] = S_ref[:, n // 2, :, NS * (n % 2):NS * (n % 2) + NS]


def kernel(x, W_router, W_kv, W_beta, b_beta, W_q):
    Tlen, batch, dim = x.shape
    out, sfin = pl.pallas_call(
        _scan_kernel,
        grid=(Tlen // CHUNK,),
        in_specs=[
            pl.BlockSpec((CHUNK, B, DIM), lambda i: (i, 0, 0)),
            pl.BlockSpec((NB, DIM), lambda i: (0, 0)),
            pl.BlockSpec((NB * 2 * NS, DIM), lambda i: (0, 0)),
            pl.BlockSpec((NB * NS, DIM), lambda i: (0, 0)),
            pl.BlockSpec((NB, NS), lambda i: (0, 0)),
            pl.BlockSpec((NS, DIM), lambda i: (0, 0)),
        ],
        out_specs=[
            pl.BlockSpec((CHUNK, B, NS), lambda i: (i, 0, 0)),
            pl.BlockSpec((B, NB, NS, NS), lambda i: (0, 0, 0, 0)),
        ],
        out_shape=[
            jax.ShapeDtypeStruct((Tlen, B, NS), jnp.float32),
            jax.ShapeDtypeStruct((B, NB, NS, NS), jnp.float32),
        ],
        scratch_shapes=[
            pltpu.VMEM((B, NB // 2, NS, 2 * NS), jnp.float32),
            pltpu.VMEM((CHUNK, B, NB, 2 * NS), jnp.float32),
            pltpu.VMEM((CHUNK, B, NB, NS), jnp.float32),
            pltpu.VMEM((CHUNK, B, NS), jnp.float32),
            pltpu.VMEM((CHUNK, B, NB // 2), jnp.float32),
            pltpu.VMEM((CHUNK, B, NB // 2), jnp.float32),
            pltpu.VMEM((CHUNK, B), jnp.int32),
            pltpu.VMEM((CHUNK, B), jnp.int32),
            pltpu.VMEM((CHUNK, B), jnp.float32),
            pltpu.VMEM((CHUNK, B), jnp.float32),
        ],
    )(x, W_router, W_kv, W_beta, b_beta, W_q)
    return out, sfin


# speedup vs baseline: 1.2442x; 1.0016x over previous
import jax
import jax.numpy as jnp
from jax.experimental import pallas as pl
from jax.experimental.pallas import tpu as pltpu

DIM = 1024
NS = 64
NB = 16
B = 4
CHUNK = 256


def _scan_kernel(x_ref, Wr_ref, Wkv_ref, Wb_ref, bb_ref, Wq_ref,
                 out_ref, sfin_ref,
                 S_ref, kn_ref, be_ref, q_ref, smc_ref,
                 i1_ref, i2_ref, w1_ref, w2_ref):
    i = pl.program_id(0)

    @pl.when(i == 0)
    def _init():
        S_ref[...] = jnp.zeros_like(S_ref)

    cb = CHUNK * B
    xf = x_ref[...].reshape(cb, DIM)

    logits = jax.lax.dot_general(xf, Wr_ref[...], (((1,), (1,)), ((), ())),
                                 preferred_element_type=jnp.float32)
    sm = jax.nn.softmax(logits, axis=-1)
    iota = jax.lax.broadcasted_iota(jnp.int32, (cb, NB), 1)
    i1 = jnp.argmax(logits, axis=-1)
    sel1 = iota == i1[:, None]
    i2 = jnp.argmax(jnp.where(sel1, -jnp.inf, logits), axis=-1)
    sel2 = iota == i2[:, None]
    s1 = jnp.sum(jnp.where(sel1, sm, 0.0), axis=-1)
    s2 = jnp.sum(jnp.where(sel2, sm, 0.0), axis=-1)
    denom = s1 + s2 + 1e-8
    rowc = jax.lax.broadcasted_iota(jnp.int32, (NB, NB * NS), 0)
    colc = jax.lax.broadcasted_iota(jnp.int32, (NB, NB * NS), 1)
    selC = ((colc // (2 * NS) == rowc // 2)
            & ((colc % (2 * NS)) // NS == rowc % 2)).astype(jnp.float32)
    smc2d = jax.lax.dot_general(sm, selC, (((1,), (0,)), ((), ())),
                                preferred_element_type=jnp.float32)
    for m in range(NB // 2):
        smc_ref[:, :, m, :] = (
            smc2d[:, 2 * NS * m:2 * NS * (m + 1)].reshape(CHUNK, B, 2 * NS))
    i1_ref[...] = i1.astype(jnp.int32).reshape(CHUNK, B)
    i2_ref[...] = i2.astype(jnp.int32).reshape(CHUNK, B)
    w1_ref[...] = (s1 / denom).reshape(CHUNK, B)
    w2_ref[...] = (s2 / denom).reshape(CHUNK, B)

    kv = jax.lax.dot_general(xf, Wkv_ref[...], (((1,), (1,)), ((), ())),
                             preferred_element_type=jnp.float32)
    bm = jax.lax.dot_general(xf, Wb_ref[...], (((1,), (1,)), ((), ())),
                             preferred_element_type=jnp.float32)
    for n in range(NB):
        k_n = kv[:, 128 * n:128 * n + NS]
        nrm = jnp.sqrt(jnp.sum(k_n * k_n, axis=-1, keepdims=True)) + 1e-6
        knv = jnp.concatenate([k_n / nrm, kv[:, 128 * n + NS:128 * n + 128]], -1)
        kn_ref[:, :, n, :] = knv.reshape(CHUNK, B, 2 * NS)
        b_n = bm[:, NS * n:NS * n + NS] + bb_ref[n][None, :]
        be_ref[:, :, n, :] = jax.nn.sigmoid(b_n).reshape(CHUNK, B, NS)
    q = jax.lax.dot_general(xf, Wq_ref[...], (((1,), (1,)), ((), ())),
                            preferred_element_type=jnp.float32)
    q_ref[...] = jnp.concatenate([q, q], -1).reshape(CHUNK, B, 2 * NS)

    lane = jax.lax.broadcasted_iota(jnp.int32, (NS, 2 * NS), 1)

    def body(t, carry):
        slots = []
        for b in range(B):
            slots.append((b, i1_ref[t, b], w1_ref[t, b]))
            slots.append((b, i2_ref[t, b], w2_ref[t, b]))
        pair8 = [S_ref[b, n // 2] for b, n, _ in slots]
        S8 = jnp.stack([jnp.where((n % 2) == 0, p[:, :NS], p[:, NS:])
                        for p, (_, n, _) in zip(pair8, slots)])
        knv8 = [kn_ref[t, b, pl.ds(n, 1)] for b, n, _ in slots]
        kn8 = jnp.stack([kv_[:, :NS] for kv_ in knv8])
        v8 = jnp.stack([kv_[:, NS:] for kv_ in knv8])
        be8 = jnp.stack([be_ref[t, b, pl.ds(n, 1)] for b, n, _ in slots])
        retr = jnp.sum(S8 * kn8, axis=-1, keepdims=True)
        delta = jnp.swapaxes(v8, -1, -2) - retr
        S_upd = jnp.tanh(jnp.swapaxes(be8, -1, -2) * S8 + delta * kn8)
        w8 = jnp.stack([jnp.full((1, 1), 1.0, jnp.float32) * w
                        for _, _, w in slots])
        S_new8 = S8 + w8 * (S_upd - S8)
        for b in range(B):
            k1, k2 = 2 * b, 2 * b + 1
            n1, n2 = slots[k1][1], slots[k2][1]
            m1, m2 = n1 // 2, n2 // 2
            msk1 = (lane < NS) == ((n1 % 2) == 0)
            msk2 = (lane < NS) == ((n2 % 2) == 0)
            wp1 = jnp.where(msk1,
                            jnp.concatenate([S_new8[k1], S_new8[k1]], -1),
                            pair8[k1])
            S_ref[b, m1] = wp1
            base2 = jnp.where(m1 == m2, wp1, pair8[k2])
            wp2 = jnp.where(msk2,
                            jnp.concatenate([S_new8[k2], S_new8[k2]], -1),
                            base2)
            S_ref[b, m2] = wp2
        S = S_ref[...]
        qq = q_ref[t]
        P = S * qq[:, None, None, :]
        z_lo = jnp.sum(P[..., :NS], axis=-1)
        z_hi = jnp.sum(P[..., NS:], axis=-1)
        zc = jnp.concatenate([z_lo, z_hi], -1)
        bo = zc * zc * jax.nn.sigmoid(zc)
        ws = jnp.sum(bo * smc_ref[t], axis=1)
        out_ref[t] = ws[:, :NS] + ws[:, NS:]
        return carry

    jax.lax.fori_loop(0, CHUNK, body, 0)

    @pl.when(i == pl.num_programs(0) - 1)
    def _fin():
        for n in range(NB):
            sfin_ref[:, n] = S_ref[:, n // 2, :, NS * (n % 2):NS * (n % 2) + NS]


def kernel(x, W_router, W_kv, W_beta, b_beta, W_q):
    Tlen, batch, dim = x.shape
    out, sfin = pl.pallas_call(
        _scan_kernel,
        grid=(Tlen // CHUNK,),
        in_specs=[
            pl.BlockSpec((CHUNK, B, DIM), lambda i: (i, 0, 0)),
            pl.BlockSpec((NB, DIM), lambda i: (0, 0)),
            pl.BlockSpec((NB * 2 * NS, DIM), lambda i: (0, 0)),
            pl.BlockSpec((NB * NS, DIM), lambda i: (0, 0)),
            pl.BlockSpec((NB, NS), lambda i: (0, 0)),
            pl.BlockSpec((NS, DIM), lambda i: (0, 0)),
        ],
        out_specs=[
            pl.BlockSpec((CHUNK, B, NS), lambda i: (i, 0, 0)),
            pl.BlockSpec((B, NB, NS, NS), lambda i: (0, 0, 0, 0)),
        ],
        out_shape=[
            jax.ShapeDtypeStruct((Tlen, B, NS), jnp.float32),
            jax.ShapeDtypeStruct((B, NB, NS, NS), jnp.float32),
        ],
        scratch_shapes=[
            pltpu.VMEM((B, NB // 2, NS, 2 * NS), jnp.float32),
            pltpu.VMEM((CHUNK, B, NB, 2 * NS), jnp.float32),
            pltpu.VMEM((CHUNK, B, NB, NS), jnp.float32),
            pltpu.VMEM((CHUNK, B, 2 * NS), jnp.float32),
            pltpu.VMEM((CHUNK, B, NB // 2, 2 * NS), jnp.float32),
            pltpu.VMEM((CHUNK, B), jnp.int32),
            pltpu.VMEM((CHUNK, B), jnp.int32),
            pltpu.VMEM((CHUNK, B), jnp.float32),
            pltpu.VMEM((CHUNK, B), jnp.float32),
        ],
    )(x, W_router, W_kv, W_beta, b_beta, W_q)
    return out, sfin
